# baseline (speedup 1.0000x reference)
import functools

import jax
import jax.numpy as jnp
from jax import lax
from jax.experimental import pallas as pl
from jax.experimental.pallas import tpu as pltpu
from jax.experimental.pallas import tpu_sc as plsc

N = 10000
E = 320000
D = 128
NORM_FACTOR = 100.0
COORDS_RANGE = 15.0
NORM_CONSTANT = 1.0

NC = 2
NS = 16
NW = NC * NS
EPW = E // NW
C = 80
NCH = EPW // C
RPT = N // NS

EB = 5000
NB = 1000


def _silu(v):
    return v * jax.nn.sigmoid(v)


def _make_gather():
    mesh = plsc.VectorSubcoreMesh(core_axis_name="c", subcore_axis_name="s")

    @functools.partial(
        pl.kernel,
        out_type=[jax.ShapeDtypeStruct((E, D), jnp.uint32),
                  jax.ShapeDtypeStruct((E, D), jnp.uint32)],
        mesh=mesh,
        scratch_types=[pltpu.VMEM((EPW,), jnp.int32),
                       pltpu.VMEM((EPW,), jnp.int32),
                       pltpu.VMEM((C, D), jnp.uint32),
                       pltpu.VMEM((C, D), jnp.uint32),
                       pltpu.VMEM((C, D), jnp.uint32),
                       pltpu.VMEM((C, D), jnp.uint32),
                       pltpu.SemaphoreType.DMA,
                       pltpu.SemaphoreType.DMA],
    )
    def gk(tab, rowi, coli, out_r, out_c, ira, ica, br0, bc0, br1, bc1, g0, g1):
        wid = lax.axis_index("s") * NC + lax.axis_index("c")
        base0 = wid * EPW
        pltpu.sync_copy(rowi.at[pl.ds(base0, EPW)], ira)
        pltpu.sync_copy(coli.at[pl.ds(base0, EPW)], ica)

        def issue(j, br, bc, sem):
            pltpu.async_copy(tab.at[ira.at[pl.ds(j * C, C)]], br, sem)
            pltpu.async_copy(tab.at[ica.at[pl.ds(j * C, C)]], bc, sem)

        def wait(j, br, bc, sem):
            pltpu.make_async_copy(tab.at[ira.at[pl.ds(j * C, C)]], br, sem).wait()
            pltpu.make_async_copy(tab.at[ica.at[pl.ds(j * C, C)]], bc, sem).wait()

        def wb(j, br, bc):
            pltpu.sync_copy(br, out_r.at[pl.ds(base0 + j * C, C)])
            pltpu.sync_copy(bc, out_c.at[pl.ds(base0 + j * C, C)])

        issue(0, br0, bc0, g0)

        def body(i, carry):
            a = 2 * i
            issue(a + 1, br1, bc1, g1)
            wait(a, br0, bc0, g0)
            wb(a, br0, bc0)
            issue(a + 2, br0, bc0, g0)
            wait(a + 1, br1, bc1, g1)
            wb(a + 1, br1, bc1)
            return carry

        lax.fori_loop(0, (NCH - 1) // 2, body, 0)
        wait(NCH - 1, br0, bc0, g0)
        wb(NCH - 1, br0, bc0)

    return gk


def _unpack_lo(g):
    return jax.lax.bitcast_convert_type(g << 16, jnp.float32)


def _unpack_hi(g):
    return jax.lax.bitcast_convert_type(g & jnp.uint32(0xFFFF0000), jnp.float32)


def _pack_pairs(a_f32, b_f32):
    au = jax.lax.bitcast_convert_type(a_f32, jnp.uint32) >> 16
    bu = jax.lax.bitcast_convert_type(b_f32, jnp.uint32) & jnp.uint32(0xFFFF0000)
    return au | bu


def _make_scatter(CH, npass):
    mesh = plsc.VectorSubcoreMesh(core_axis_name="c", subcore_axis_name="s")

    @functools.partial(
        pl.kernel,
        out_type=jax.ShapeDtypeStruct((NC * N, D), jnp.float32),
        mesh=mesh,
        scratch_types=[pltpu.VMEM((C,), jnp.int32),
                       pltpu.VMEM((C,), jnp.int32),
                       pltpu.VMEM((C, CH), jnp.float32),
                       pltpu.VMEM((C, CH), jnp.float32),
                       pltpu.VMEM((RPT, D), jnp.float32),
                       pltpu.VMEM_SHARED((N, CH), jnp.float32),
                       pltpu.SemaphoreType.DMA,
                       pltpu.SemaphoreType.DMA,
                       pltpu.SemaphoreType.DMA,
                       pltpu.SemaphoreType.DMA],
        compiler_params=pltpu.CompilerParams(use_tc_tiling_on_sc=False),
    )
    def sk(vals, rowi, zer, out, ib0, ib1, vb0, vb1, zb, acc, l0, l1, s0, s1):
        cid = lax.axis_index("c")
        sid = lax.axis_index("s")
        wid = cid * NS + sid
        base0 = wid * EPW
        myrows = pl.ds(sid * RPT, RPT)
        for ch in range(npass):
            cols = pl.ds(ch * CH, CH)
            pltpu.sync_copy(zer, zb)
            pltpu.sync_copy(zb.at[:, pl.ds(0, CH)], acc.at[myrows])
            plsc.subcore_barrier()

            def load(j, ib, vb, sem):
                pltpu.async_copy(rowi.at[pl.ds(base0 + j * C, C)], ib, sem)
                pltpu.async_copy(vals.at[pl.ds(base0 + j * C, C), cols], vb, sem)

            def wait_load(j, ib, vb, sem):
                pltpu.make_async_copy(
                    rowi.at[pl.ds(base0 + j * C, C)], ib, sem).wait()
                pltpu.make_async_copy(
                    vals.at[pl.ds(base0 + j * C, C), cols], vb, sem).wait()

            def scat(ib, vb, sem):
                pltpu.async_copy(vb, acc.at[ib], sem, add=True)

            def wait_scat(ib, vb, sem):
                pltpu.make_async_copy(vb, acc.at[ib], sem).wait()

            load(0, ib0, vb0, l0)

            def body(i, carry):
                a = 2 * i
                wait_load(a, ib0, vb0, l0)
                scat(ib0, vb0, s0)

                @pl.when(i > 0)
                def _():
                    wait_scat(ib1, vb1, s1)

                load(a + 1, ib1, vb1, l1)
                wait_load(a + 1, ib1, vb1, l1)
                scat(ib1, vb1, s1)
                wait_scat(ib0, vb0, s0)
                load(a + 2, ib0, vb0, l0)
                return carry

            lax.fori_loop(0, (NCH - 1) // 2, body, 0)
            wait_load(NCH - 1, ib0, vb0, l0)
            scat(ib0, vb0, s0)
            wait_scat(ib1, vb1, s1)
            wait_scat(ib0, vb0, s0)
            plsc.subcore_barrier()
            pltpu.sync_copy(acc.at[myrows], zb.at[:, pl.ds(0, CH)])
            pltpu.sync_copy(zb.at[:, pl.ds(0, CH)],
                            out.at[pl.ds(cid * N + sid * RPT, RPT), cols])

    return sk


def _edge1_body(gr, gc, ea, w1ra, w1rb, w1ca, w1cb, w1e, b1, w2, b2,
                watt, batt, m_out, cdr_out):
    g_r = gr[...]
    g_c = gc[...]
    rA = _unpack_lo(g_r[:, :64])
    rB = _unpack_hi(g_r[:, :64])
    cA = _unpack_lo(g_c[:, :64])
    cB = _unpack_hi(g_c[:, :64])
    rx = jax.lax.bitcast_convert_type(g_r[:, 64:67], jnp.float32)
    cx = jax.lax.bitcast_convert_type(g_c[:, 64:67], jnp.float32)
    cd = rx - cx
    radial = jnp.sum(cd * cd, axis=1, keepdims=True)
    cdn = cd / (jnp.sqrt(radial + 1e-8) + NORM_CONSTANT)
    lane0 = (lax.broadcasted_iota(jnp.int32, (1, 8), 1) == 0).astype(jnp.float32)
    eap = ea[...] + radial * lane0
    pre = (jnp.dot(rA, w1ra[...], preferred_element_type=jnp.float32)
           + jnp.dot(rB, w1rb[...], preferred_element_type=jnp.float32)
           + jnp.dot(cA, w1ca[...], preferred_element_type=jnp.float32)
           + jnp.dot(cB, w1cb[...], preferred_element_type=jnp.float32)
           + jnp.dot(eap, w1e[...], preferred_element_type=jnp.float32)
           + b1[...])
    m = _silu(pre)
    m = _silu(jnp.dot(m, w2[...], preferred_element_type=jnp.float32) + b2[...])
    att = jax.nn.sigmoid(jnp.sum(m * watt[...], axis=1, keepdims=True) + batt[...])
    m_out[...] = m * att
    cdr_out[...] = jnp.concatenate([cdn, radial], axis=1)


def _edge2_body(g2r, g2c, ea, cdr, w1ra, w1rb, w1ca, w1cb, w1e, b1, w2, b2,
                wc3, t_out):
    radial = cdr[:, 3:4]
    cdn = cdr[:, 0:3]
    rA = _unpack_lo(g2r[:, :64])
    rB = _unpack_hi(g2r[:, :64])
    cA = _unpack_lo(g2c[:, :64])
    cB = _unpack_hi(g2c[:, :64])
    lane0 = (lax.broadcasted_iota(jnp.int32, (1, 8), 1) == 0).astype(jnp.float32)
    eap = ea[...] + radial * lane0
    pre = (jnp.dot(rA, w1ra[...], preferred_element_type=jnp.float32)
           + jnp.dot(rB, w1rb[...], preferred_element_type=jnp.float32)
           + jnp.dot(cA, w1ca[...], preferred_element_type=jnp.float32)
           + jnp.dot(cB, w1cb[...], preferred_element_type=jnp.float32)
           + jnp.dot(eap, w1e[...], preferred_element_type=jnp.float32)
           + b1[...])
    s = _silu(pre)
    s = _silu(jnp.dot(s, w2[...], preferred_element_type=jnp.float32) + b2[...])
    t = jnp.sum(s * wc3[...], axis=1, keepdims=True)
    tr = cdn * jnp.tanh(t) * COORDS_RANGE
    t_out[...] = jnp.concatenate(
        [tr, jnp.zeros((tr.shape[0], 125), jnp.float32)], axis=1)


def _node_body(h, p0, p1, wa, wb, b1, w2, b2, out, out_bf):
    agg = (p0[...] + p1[...]) * (1.0 / NORM_FACTOR)
    pre = (jnp.dot(h[...], wa[...], preferred_element_type=jnp.float32)
           + jnp.dot(agg, wb[...], preferred_element_type=jnp.float32)
           + b1[...])
    u = _silu(pre)
    hn = h[...] + jnp.dot(u, w2[...], preferred_element_type=jnp.float32) + b2[...]
    out[...] = hn
    out_bf[...] = jnp.concatenate(
        [_pack_pairs(hn[:, :64], hn[:, 64:]),
         jnp.zeros((hn.shape[0], 64), jnp.uint32)], axis=1)


def _x_body(x, q0, q1, out):
    out[...] = x[...] + (q0[:, 0:3] + q1[:, 0:3]) * (1.0 / NORM_FACTOR)


def _blk(shape, pos=0):
    if pos is None:
        return pl.BlockSpec(shape, lambda i: (0,) * len(shape))
    return pl.BlockSpec(shape, lambda i: (i,) + (0,) * (len(shape) - 1))


def _edge1_call(gr, gc, eap, w1ra, w1rb, w1ca, w1cb, w1e, b1, w2, b2, watt, batt):
    grid = (E // EB,)
    return pl.pallas_call(
        _edge1_body,
        grid=grid,
        in_specs=[_blk((EB, D)), _blk((EB, D)), _blk((EB, 8)),
                  _blk((64, D), None), _blk((64, D), None),
                  _blk((64, D), None), _blk((64, D), None), _blk((8, D), None),
                  _blk((1, D), None), _blk((D, D), None), _blk((1, D), None),
                  _blk((1, D), None), _blk((1, 1), None)],
        out_specs=[_blk((EB, D)), _blk((EB, 4))],
        out_shape=[jax.ShapeDtypeStruct((E, D), jnp.float32),
                   jax.ShapeDtypeStruct((E, 4), jnp.float32)],
    )(gr, gc, eap, w1ra, w1rb, w1ca, w1cb, w1e, b1, w2, b2, watt, batt)


def _edge2_call(g2r, g2c, eap, cdr, w1ra, w1rb, w1ca, w1cb, w1e, b1, w2, b2, wc3):
    grid = (E // EB,)
    return pl.pallas_call(
        _edge2_body,
        grid=grid,
        in_specs=[_blk((EB, D)), _blk((EB, D)), _blk((EB, 8)), _blk((EB, 4)),
                  _blk((64, D), None), _blk((64, D), None),
                  _blk((64, D), None), _blk((64, D), None), _blk((8, D), None),
                  _blk((1, D), None), _blk((D, D), None), _blk((1, D), None),
                  _blk((1, D), None)],
        out_specs=_blk((EB, D)),
        out_shape=jax.ShapeDtypeStruct((E, D), jnp.float32),
    )(g2r, g2c, eap, cdr, w1ra, w1rb, w1ca, w1cb, w1e, b1, w2, b2, wc3)


def _node_call(h, p0, p1, wa, wb, b1, w2, b2):
    grid = (N // NB,)
    return pl.pallas_call(
        _node_body,
        grid=grid,
        in_specs=[_blk((NB, D)), _blk((NB, D)), _blk((NB, D)),
                  _blk((D, D), None), _blk((D, D), None), _blk((1, D), None),
                  _blk((D, D), None), _blk((1, D), None)],
        out_specs=[_blk((NB, D)), _blk((NB, D))],
        out_shape=[jax.ShapeDtypeStruct((N, D), jnp.float32),
                   jax.ShapeDtypeStruct((N, D), jnp.uint32)],
    )(h, p0, p1, wa, wb, b1, w2, b2)


def _x_call(x, q0, q1):
    grid = (N // NB,)
    return pl.pallas_call(
        _x_body,
        grid=grid,
        in_specs=[_blk((NB, 3)), _blk((NB, D)), _blk((NB, D))],
        out_specs=_blk((NB, 3)),
        out_shape=jax.ShapeDtypeStruct((N, 3), jnp.float32),
    )(x, q0, q1)


def kernel(h, x, edge_index, edge_attr,
           W_m1, b_m1, W_m2, b_m2, W_att, b_att, W_u1, b_u1, W_u2, b_u2,
           W_c1, b_c1, W_c2, b_c2, W_c3):
    row = edge_index[0]
    col = edge_index[1]
    eap = jnp.pad(edge_attr, ((0, 0), (1, 3)))
    zer = jnp.zeros((RPT, D), jnp.float32)

    T1 = jnp.concatenate(
        [_pack_pairs(h[:, :64], h[:, 64:]),
         jax.lax.bitcast_convert_type(x, jnp.uint32),
         jnp.zeros((N, 61), jnp.uint32)], axis=1)
    gr, gc = _make_gather()(T1, row, col)
    w1e = jnp.pad(W_m1[2 * D:], ((0, 3), (0, 0)))

    m, cdr = _edge1_call(gr, gc, eap,
                         W_m1[:64], W_m1[64:D],
                         W_m1[D:D + 64], W_m1[D + 64:2 * D],
                         w1e, b_m1.reshape(1, D),
                         W_m2, b_m2.reshape(1, D),
                         W_att.reshape(1, D), b_att.reshape(1, 1))
    part = _make_scatter(64, 2)(m, row, zer)

    hn, hn_pk = _node_call(h, part[:N], part[N:],
                           W_u1[:D], W_u1[D:], b_u1.reshape(1, D),
                           W_u2, b_u2.reshape(1, D))

    g2r, g2c = _make_gather()(hn_pk, row, col)
    wc1e = jnp.pad(W_c1[2 * D:], ((0, 3), (0, 0)))
    trans = _edge2_call(g2r, g2c, eap, cdr,
                        W_c1[:64], W_c1[64:D],
                        W_c1[D:D + 64], W_c1[D + 64:2 * D],
                        wc1e, b_c1.reshape(1, D),
                        W_c2, b_c2.reshape(1, D),
                        W_c3.reshape(1, D))
    q = _make_scatter(16, 1)(trans, row, zer)
    xn = _x_call(x, q[:N], q[N:])
    return (hn, xn)


# speedup vs baseline: 1.1547x; 1.1547x over previous
import functools

import jax
import jax.numpy as jnp
from jax import lax
from jax.experimental import pallas as pl
from jax.experimental.pallas import tpu as pltpu
from jax.experimental.pallas import tpu_sc as plsc

N = 10000
E = 320000
D = 128
NORM_FACTOR = 100.0
COORDS_RANGE = 15.0
NORM_CONSTANT = 1.0

NC = 2
NS = 16
NW = NC * NS
EPW = E // NW
C = 80
NCH = EPW // C
RPT = N // NS

EB = 4000
NB = 1000


def _silu(v):
    return v * jax.nn.sigmoid(v)


def _make_gather():
    mesh = plsc.VectorSubcoreMesh(core_axis_name="c", subcore_axis_name="s")

    @functools.partial(
        pl.kernel,
        out_type=[jax.ShapeDtypeStruct((E, D), jnp.uint32),
                  jax.ShapeDtypeStruct((E, D), jnp.uint32)],
        mesh=mesh,
        scratch_types=[pltpu.VMEM((EPW,), jnp.int32),
                       pltpu.VMEM((EPW,), jnp.int32),
                       pltpu.VMEM((C, D), jnp.uint32),
                       pltpu.VMEM((C, D), jnp.uint32),
                       pltpu.VMEM((C, D), jnp.uint32),
                       pltpu.VMEM((C, D), jnp.uint32),
                       pltpu.SemaphoreType.DMA,
                       pltpu.SemaphoreType.DMA],
    )
    def gk(tab, rowi, coli, out_r, out_c, ira, ica, br0, bc0, br1, bc1, g0, g1):
        wid = lax.axis_index("s") * NC + lax.axis_index("c")
        base0 = wid * EPW
        pltpu.sync_copy(rowi.at[pl.ds(base0, EPW)], ira)
        pltpu.sync_copy(coli.at[pl.ds(base0, EPW)], ica)

        def issue(j, br, bc, sem):
            pltpu.async_copy(tab.at[ira.at[pl.ds(j * C, C)]], br, sem)
            pltpu.async_copy(tab.at[ica.at[pl.ds(j * C, C)]], bc, sem)

        def wait(j, br, bc, sem):
            pltpu.make_async_copy(tab.at[ira.at[pl.ds(j * C, C)]], br, sem).wait()
            pltpu.make_async_copy(tab.at[ica.at[pl.ds(j * C, C)]], bc, sem).wait()

        def wb(j, br, bc):
            pltpu.sync_copy(br, out_r.at[pl.ds(base0 + j * C, C)])
            pltpu.sync_copy(bc, out_c.at[pl.ds(base0 + j * C, C)])

        issue(0, br0, bc0, g0)

        def body(i, carry):
            a = 2 * i
            issue(a + 1, br1, bc1, g1)
            wait(a, br0, bc0, g0)
            wb(a, br0, bc0)
            issue(a + 2, br0, bc0, g0)
            wait(a + 1, br1, bc1, g1)
            wb(a + 1, br1, bc1)
            return carry

        lax.fori_loop(0, (NCH - 1) // 2, body, 0)
        wait(NCH - 1, br0, bc0, g0)
        wb(NCH - 1, br0, bc0)

    return gk


def _unpack_lo(g):
    return jax.lax.bitcast_convert_type(g << 16, jnp.float32)


def _unpack_hi(g):
    return jax.lax.bitcast_convert_type(g & jnp.uint32(0xFFFF0000), jnp.float32)


def _pack_pairs(a_f32, b_f32):
    au = jax.lax.bitcast_convert_type(a_f32, jnp.uint32) >> 16
    bu = jax.lax.bitcast_convert_type(b_f32, jnp.uint32) & jnp.uint32(0xFFFF0000)
    return au | bu


def _make_scatter(CH, npass):
    mesh = plsc.VectorSubcoreMesh(core_axis_name="c", subcore_axis_name="s")

    @functools.partial(
        pl.kernel,
        out_type=jax.ShapeDtypeStruct((NC * N, D), jnp.float32),
        mesh=mesh,
        scratch_types=[pltpu.VMEM((C,), jnp.int32),
                       pltpu.VMEM((C,), jnp.int32),
                       pltpu.VMEM((C, CH), jnp.float32),
                       pltpu.VMEM((C, CH), jnp.float32),
                       pltpu.VMEM((RPT, D), jnp.float32),
                       pltpu.VMEM_SHARED((N, CH), jnp.float32),
                       pltpu.SemaphoreType.DMA,
                       pltpu.SemaphoreType.DMA,
                       pltpu.SemaphoreType.DMA,
                       pltpu.SemaphoreType.DMA],
        compiler_params=pltpu.CompilerParams(use_tc_tiling_on_sc=False),
    )
    def sk(vals, rowi, zer, out, ib0, ib1, vb0, vb1, zb, acc, l0, l1, s0, s1):
        cid = lax.axis_index("c")
        sid = lax.axis_index("s")
        wid = cid * NS + sid
        base0 = wid * EPW
        myrows = pl.ds(sid * RPT, RPT)
        for ch in range(npass):
            cols = pl.ds(ch * CH, CH)
            pltpu.sync_copy(zer, zb)
            pltpu.sync_copy(zb.at[:, pl.ds(0, CH)], acc.at[myrows])
            plsc.subcore_barrier()

            def load(j, ib, vb, sem):
                pltpu.async_copy(rowi.at[pl.ds(base0 + j * C, C)], ib, sem)
                pltpu.async_copy(vals.at[pl.ds(base0 + j * C, C), cols], vb, sem)

            def wait_load(j, ib, vb, sem):
                pltpu.make_async_copy(
                    rowi.at[pl.ds(base0 + j * C, C)], ib, sem).wait()
                pltpu.make_async_copy(
                    vals.at[pl.ds(base0 + j * C, C), cols], vb, sem).wait()

            def scat(ib, vb, sem):
                pltpu.async_copy(vb, acc.at[ib], sem, add=True)

            def wait_scat(ib, vb, sem):
                pltpu.make_async_copy(vb, acc.at[ib], sem).wait()

            load(0, ib0, vb0, l0)

            def body(i, carry):
                a = 2 * i
                wait_load(a, ib0, vb0, l0)
                scat(ib0, vb0, s0)

                @pl.when(i > 0)
                def _():
                    wait_scat(ib1, vb1, s1)

                load(a + 1, ib1, vb1, l1)
                wait_load(a + 1, ib1, vb1, l1)
                scat(ib1, vb1, s1)
                wait_scat(ib0, vb0, s0)
                load(a + 2, ib0, vb0, l0)
                return carry

            lax.fori_loop(0, (NCH - 1) // 2, body, 0)
            wait_load(NCH - 1, ib0, vb0, l0)
            scat(ib0, vb0, s0)
            wait_scat(ib1, vb1, s1)
            wait_scat(ib0, vb0, s0)
            plsc.subcore_barrier()
            pltpu.sync_copy(acc.at[myrows], zb.at[:, pl.ds(0, CH)])
            pltpu.sync_copy(zb.at[:, pl.ds(0, CH)],
                            out.at[pl.ds(cid * N + sid * RPT, RPT), cols])

    return sk


def _edge1_body(gr, gc, ea, w1ra, w1rb, w1ca, w1cb, w1e, b1, w2, b2,
                watt, batt, m_out, cdr_out):
    g_r = gr[...]
    g_c = gc[...]
    rA = _unpack_lo(g_r[:, :64])
    rB = _unpack_hi(g_r[:, :64])
    cA = _unpack_lo(g_c[:, :64])
    cB = _unpack_hi(g_c[:, :64])
    rx = jax.lax.bitcast_convert_type(g_r[:, 64:67], jnp.float32)
    cx = jax.lax.bitcast_convert_type(g_c[:, 64:67], jnp.float32)
    cd = rx - cx
    radial = jnp.sum(cd * cd, axis=1, keepdims=True)
    cdn = cd / (jnp.sqrt(radial + 1e-8) + NORM_CONSTANT)
    lane0 = (lax.broadcasted_iota(jnp.int32, (1, 8), 1) == 0).astype(jnp.float32)
    eap = ea[...] + radial * lane0
    pre = (jnp.dot(rA, w1ra[...], preferred_element_type=jnp.float32)
           + jnp.dot(rB, w1rb[...], preferred_element_type=jnp.float32)
           + jnp.dot(cA, w1ca[...], preferred_element_type=jnp.float32)
           + jnp.dot(cB, w1cb[...], preferred_element_type=jnp.float32)
           + jnp.dot(eap, w1e[...], preferred_element_type=jnp.float32)
           + b1[...])
    m = _silu(pre)
    m = _silu(jnp.dot(m, w2[...], preferred_element_type=jnp.float32) + b2[...])
    att = jax.nn.sigmoid(jnp.sum(m * watt[...], axis=1, keepdims=True) + batt[...])
    m_out[...] = m * att
    cdr_out[...] = jnp.concatenate([cdn, radial], axis=1)


def _edge2_body(g2r, g2c, ea, cdr, w1ra, w1rb, w1ca, w1cb, w1e, b1, w2, b2,
                wc3, t_out):
    radial = cdr[:, 3:4]
    cdn = cdr[:, 0:3]
    rA = _unpack_lo(g2r[:, :64])
    rB = _unpack_hi(g2r[:, :64])
    cA = _unpack_lo(g2c[:, :64])
    cB = _unpack_hi(g2c[:, :64])
    lane0 = (lax.broadcasted_iota(jnp.int32, (1, 8), 1) == 0).astype(jnp.float32)
    eap = ea[...] + radial * lane0
    pre = (jnp.dot(rA, w1ra[...], preferred_element_type=jnp.float32)
           + jnp.dot(rB, w1rb[...], preferred_element_type=jnp.float32)
           + jnp.dot(cA, w1ca[...], preferred_element_type=jnp.float32)
           + jnp.dot(cB, w1cb[...], preferred_element_type=jnp.float32)
           + jnp.dot(eap, w1e[...], preferred_element_type=jnp.float32)
           + b1[...])
    s = _silu(pre)
    s = _silu(jnp.dot(s, w2[...], preferred_element_type=jnp.float32) + b2[...])
    t = jnp.sum(s * wc3[...], axis=1, keepdims=True)
    tr = cdn * jnp.tanh(t) * COORDS_RANGE
    t_out[...] = jnp.concatenate(
        [tr, jnp.zeros((tr.shape[0], 125), jnp.float32)], axis=1)


def _node_body(h, p0, p1, wa, wb, b1, w2, b2, out, out_bf):
    agg = (p0[...] + p1[...]) * (1.0 / NORM_FACTOR)
    pre = (jnp.dot(h[...], wa[...], preferred_element_type=jnp.float32)
           + jnp.dot(agg, wb[...], preferred_element_type=jnp.float32)
           + b1[...])
    u = _silu(pre)
    hn = h[...] + jnp.dot(u, w2[...], preferred_element_type=jnp.float32) + b2[...]
    out[...] = hn
    out_bf[...] = jnp.concatenate(
        [_pack_pairs(hn[:, :64], hn[:, 64:]),
         jnp.zeros((hn.shape[0], 64), jnp.uint32)], axis=1)


def _x_body(x, q0, q1, out):
    out[...] = x[...] + (q0[:, 0:3] + q1[:, 0:3]) * (1.0 / NORM_FACTOR)


def _blk(shape, pos=0):
    if pos is None:
        return pl.BlockSpec(shape, lambda i: (0,) * len(shape))
    return pl.BlockSpec(shape, lambda i: (i,) + (0,) * (len(shape) - 1))


def _edge1_call(gr, gc, eap, w1ra, w1rb, w1ca, w1cb, w1e, b1, w2, b2, watt, batt):
    grid = (E // EB,)
    return pl.pallas_call(
        _edge1_body,
        grid=grid,
        in_specs=[_blk((EB, D)), _blk((EB, D)), _blk((EB, 8)),
                  _blk((64, D), None), _blk((64, D), None),
                  _blk((64, D), None), _blk((64, D), None), _blk((8, D), None),
                  _blk((1, D), None), _blk((D, D), None), _blk((1, D), None),
                  _blk((1, D), None), _blk((1, 1), None)],
        out_specs=[_blk((EB, D)), _blk((EB, 4))],
        out_shape=[jax.ShapeDtypeStruct((E, D), jnp.float32),
                   jax.ShapeDtypeStruct((E, 4), jnp.float32)],
    )(gr, gc, eap, w1ra, w1rb, w1ca, w1cb, w1e, b1, w2, b2, watt, batt)


def _edge2_call(g2r, g2c, eap, cdr, w1ra, w1rb, w1ca, w1cb, w1e, b1, w2, b2, wc3):
    grid = (E // EB,)
    return pl.pallas_call(
        _edge2_body,
        grid=grid,
        in_specs=[_blk((EB, D)), _blk((EB, D)), _blk((EB, 8)), _blk((EB, 4)),
                  _blk((64, D), None), _blk((64, D), None),
                  _blk((64, D), None), _blk((64, D), None), _blk((8, D), None),
                  _blk((1, D), None), _blk((D, D), None), _blk((1, D), None),
                  _blk((1, D), None)],
        out_specs=_blk((EB, D)),
        out_shape=jax.ShapeDtypeStruct((E, D), jnp.float32),
    )(g2r, g2c, eap, cdr, w1ra, w1rb, w1ca, w1cb, w1e, b1, w2, b2, wc3)


def _node_call(h, p0, p1, wa, wb, b1, w2, b2):
    grid = (N // NB,)
    return pl.pallas_call(
        _node_body,
        grid=grid,
        in_specs=[_blk((NB, D)), _blk((NB, D)), _blk((NB, D)),
                  _blk((D, D), None), _blk((D, D), None), _blk((1, D), None),
                  _blk((D, D), None), _blk((1, D), None)],
        out_specs=[_blk((NB, D)), _blk((NB, D))],
        out_shape=[jax.ShapeDtypeStruct((N, D), jnp.float32),
                   jax.ShapeDtypeStruct((N, D), jnp.uint32)],
    )(h, p0, p1, wa, wb, b1, w2, b2)


def _x_call(x, q0, q1):
    grid = (N // NB,)
    return pl.pallas_call(
        _x_body,
        grid=grid,
        in_specs=[_blk((NB, 3)), _blk((NB, D)), _blk((NB, D))],
        out_specs=_blk((NB, 3)),
        out_shape=jax.ShapeDtypeStruct((N, 3), jnp.float32),
    )(x, q0, q1)


def kernel(h, x, edge_index, edge_attr,
           W_m1, b_m1, W_m2, b_m2, W_att, b_att, W_u1, b_u1, W_u2, b_u2,
           W_c1, b_c1, W_c2, b_c2, W_c3):
    row = edge_index[0]
    col = edge_index[1]
    eap = jnp.pad(edge_attr, ((0, 0), (1, 3)))
    zer = jnp.zeros((RPT, D), jnp.float32)

    T1 = jnp.concatenate(
        [_pack_pairs(h[:, :64], h[:, 64:]),
         jax.lax.bitcast_convert_type(x, jnp.uint32),
         jnp.zeros((N, 61), jnp.uint32)], axis=1)
    gr, gc = _make_gather()(T1, row, col)
    w1e = jnp.pad(W_m1[2 * D:], ((0, 3), (0, 0)))

    m, cdr = _edge1_call(gr, gc, eap,
                         W_m1[:64], W_m1[64:D],
                         W_m1[D:D + 64], W_m1[D + 64:2 * D],
                         w1e, b_m1.reshape(1, D),
                         W_m2, b_m2.reshape(1, D),
                         W_att.reshape(1, D), b_att.reshape(1, 1))
    part = _make_scatter(64, 2)(m, row, zer)

    hn, hn_pk = _node_call(h, part[:N], part[N:],
                           W_u1[:D], W_u1[D:], b_u1.reshape(1, D),
                           W_u2, b_u2.reshape(1, D))

    g2r, g2c = _make_gather()(hn_pk, row, col)
    wc1e = jnp.pad(W_c1[2 * D:], ((0, 3), (0, 0)))
    trans = _edge2_call(g2r, g2c, eap, cdr,
                        W_c1[:64], W_c1[64:D],
                        W_c1[D:D + 64], W_c1[D + 64:2 * D],
                        wc1e, b_c1.reshape(1, D),
                        W_c2, b_c2.reshape(1, D),
                        W_c3.reshape(1, D))
    q = _make_scatter(16, 1)(trans, row, zer)
    xn = _x_call(x, q[:N], q[N:])
    return (hn, xn)
